# Initial kernel scaffold; baseline (speedup 1.0000x reference)
import jax
import jax.numpy as jnp
from jax.experimental import pallas as pl


def kernel(boxes, scores):
    raise NotImplementedError("write your pallas kernel here")


# speedup vs baseline: 215.8935x; 215.8935x over previous
import jax
import jax.numpy as jnp
from jax.experimental import pallas as pl
from jax.experimental.pallas import tpu as pltpu

_N_BOXES = 5000
_LANES = 128
_ROWS = 40
_N_PAD = _ROWS * _LANES
_IOU_THRESHOLD = 0.5
_MAX_OUT = 100
_OUT_PAD = 128
_IMG_SIZE = 512.0


def _nms_body(y1_ref, x1_ref, y2_ref, x2_ref, s_ref,
              oy1, ox1, oy2, ox2, osc,
              cy1, cx1, cy2, cx2, area_ref, ms_ref):
    for r in (oy1, ox1, oy2, ox2, osc):
        r[...] = jnp.zeros_like(r)

    a = jnp.clip(y1_ref[...], 0.0, _IMG_SIZE)
    b = jnp.clip(x1_ref[...], 0.0, _IMG_SIZE)
    c = jnp.clip(y2_ref[...], 0.0, _IMG_SIZE)
    d = jnp.clip(x2_ref[...], 0.0, _IMG_SIZE)
    cy1[...] = a
    cx1[...] = b
    cy2[...] = c
    cx2[...] = d
    area_ref[...] = (c - a) * (d - b)
    ms_ref[...] = s_ref[...]

    lin = (jax.lax.broadcasted_iota(jnp.int32, (_ROWS, _LANES), 0) * _LANES
           + jax.lax.broadcasted_iota(jnp.int32, (_ROWS, _LANES), 1))
    lane = jax.lax.broadcasted_iota(jnp.int32, (1, _LANES), 1)

    def step(k, m):
        def do_iter(m):
            ms = ms_ref[...]
            jsel = jnp.min(jnp.where(ms == m, lin, _N_PAD))
            r = jsel // _LANES
            l = jsel - r * _LANES
            row_sel = lane == l

            def pick(ref):
                row = ref[pl.ds(r, 1), :]
                return jnp.sum(jnp.where(row_sel, row, 0.0))

            by1 = pick(cy1)
            bx1 = pick(cx1)
            by2 = pick(cy2)
            bx2 = pick(cx2)
            barea = (by2 - by1) * (bx2 - bx1)

            oy1[pl.ds(k, 1), :] = by1[None, None]
            ox1[pl.ds(k, 1), :] = bx1[None, None]
            oy2[pl.ds(k, 1), :] = by2[None, None]
            ox2[pl.ds(k, 1), :] = bx2[None, None]
            osc[pl.ds(k, 1), :] = m[None, None]

            iy1 = jnp.maximum(cy1[...], by1)
            ix1 = jnp.maximum(cx1[...], bx1)
            iy2 = jnp.minimum(cy2[...], by2)
            ix2 = jnp.minimum(cx2[...], bx2)
            inter = (jnp.clip(iy2 - iy1, 0.0, None)
                     * jnp.clip(ix2 - ix1, 0.0, None))
            union = area_ref[...] + barea - inter
            iou = inter / jnp.maximum(union, 1e-8)
            ms_new = jnp.where(iou > _IOU_THRESHOLD, -1.0, ms)
            ms_new = jnp.where(lin == jsel, -1.0, ms_new)
            ms_ref[...] = ms_new
            return jnp.max(ms_new)

        return jax.lax.cond(m > -0.5, do_iter, lambda mm: mm, m)

    m0 = jnp.max(ms_ref[...])
    jax.lax.fori_loop(0, _MAX_OUT, step, m0)


def _make_call(interpret=False):
    return pl.pallas_call(
        _nms_body,
        out_shape=[jax.ShapeDtypeStruct((_OUT_PAD, 1), jnp.float32)] * 5,
        scratch_shapes=[pltpu.VMEM((_ROWS, _LANES), jnp.float32)] * 6,
        interpret=interpret,
    )


def kernel(boxes, scores):
    pad = _N_PAD - _N_BOXES
    y1 = jnp.pad(boxes[:, 0], (0, pad)).reshape(_ROWS, _LANES)
    x1 = jnp.pad(boxes[:, 1], (0, pad)).reshape(_ROWS, _LANES)
    y2 = jnp.pad(boxes[:, 2], (0, pad)).reshape(_ROWS, _LANES)
    x2 = jnp.pad(boxes[:, 3], (0, pad)).reshape(_ROWS, _LANES)
    s = jnp.pad(scores, (0, pad), constant_values=-1.0).reshape(_ROWS, _LANES)
    oy1, ox1, oy2, ox2, osc = _make_call()(y1, x1, y2, x2, s)
    out = jnp.concatenate([oy1, ox1, oy2, ox2, osc], axis=1)[:_MAX_OUT]
    return out
